# baseline (speedup 1.0000x reference)
import functools

import jax
import jax.numpy as jnp
from jax import lax
from jax.experimental import pallas as pl
from jax.experimental.pallas import tpu as pltpu
from jax.experimental.pallas import tpu_sc as plsc

B = 16
S = 2048
H = 32
KVH = 8
DH = 128
SLOTS = 32768
SCALE = 0.08838834764831845
GROUP = H // KVH
ROW = KVH * DH

NTILES = 32
CH = 32
CPB = S // CH
NSPLIT = 8
NBH = B // NSPLIT
NCAND = NBH * CPB // NTILES
CPT = NCAND // NBH
OFF = NTILES // NBH

SBLK = 256
NSB = S // SBLK

NEG = -1e30


def _sc_gather_body(b0, kc_hbm, vc_hbm, asl_hbm, lens_hbm, gk_hbm, gv_hbm,
                    idx_v, kbuf, vbuf, lens_v, sem_k, sem_v, sem_wk, sem_wv):
    t = lax.axis_index("s") * 2 + lax.axis_index("c")
    pltpu.sync_copy(lens_hbm, lens_v)
    lane = lax.iota(jnp.int32, 16)

    def body(i, carry):
        bb = i // CPT
        j = i % CPT
        c = lax.rem(t + NTILES - OFF * bb, NTILES) + NTILES * j
        b = b0 + bb
        lenb = jnp.sum(jnp.where(lane == b, lens_v[...], 0.0), axis=0)
        active = (c == 0) | (lenb > (c * CH).astype(jnp.float32))

        @pl.when(active)
        def _():
            s0 = c * CH
            pltpu.sync_copy(asl_hbm.at[pl.ds(b * S + s0, CH)], idx_v)
            ck = pltpu.async_copy(kc_hbm.at[idx_v], kbuf, sem_k)
            cv = pltpu.async_copy(vc_hbm.at[idx_v], vbuf, sem_v)
            ck.wait()
            wk = [pltpu.async_copy(kbuf.at[:, h, :],
                                   gk_hbm.at[bb, h, pl.ds(s0, CH), :],
                                   sem_wk)
                  for h in range(KVH)]
            cv.wait()
            wv = [pltpu.async_copy(vbuf.at[:, h, :],
                                   gv_hbm.at[bb, h, pl.ds(s0, CH), :],
                                   sem_wv)
                  for h in range(KVH)]
            for w in wk:
                w.wait()
            for w in wv:
                w.wait()

        return carry

    lax.fori_loop(0, NCAND, body, 0)


def _sc_gather(b0, kc, vc, asl_flat, lens_f):
    mesh = plsc.VectorSubcoreMesh(core_axis_name="c", subcore_axis_name="s")
    fn = pl.kernel(
        functools.partial(_sc_gather_body, b0),
        out_type=[jax.ShapeDtypeStruct((NBH, KVH, S, DH), jnp.float32),
                  jax.ShapeDtypeStruct((NBH, KVH, S, DH), jnp.float32)],
        mesh=mesh,
        scratch_types=[
            pltpu.VMEM((CH,), jnp.int32),
            pltpu.VMEM((CH, KVH, DH), jnp.float32),
            pltpu.VMEM((CH, KVH, DH), jnp.float32),
            pltpu.VMEM((B,), jnp.float32),
            pltpu.SemaphoreType.DMA,
            pltpu.SemaphoreType.DMA,
            pltpu.SemaphoreType.DMA,
            pltpu.SemaphoreType.DMA,
        ],
        compiler_params=pltpu.CompilerParams(needs_layout_passes=False),
        name=f"sc_gather_b{b0}",
    )
    return fn(kc, vc, asl_flat, lens_f)


def _tc_attend_body(lens_sref, q_ref, kn_ref, vn_ref, sm_ref, asl_ref,
                    gk_ref, gv_ref, out_ref, m_s, l_s, acc_s, c_s):
    b = pl.program_id(0)
    sb = pl.program_id(1)
    lenb = jnp.maximum(lens_sref[b], 1)
    nact = (lenb + SBLK - 1) // SBLK

    @pl.when(sb == 0)
    def _():
        m_s[...] = jnp.full((H, 1), -3e38, jnp.float32)
        l_s[...] = jnp.zeros((H, 1), jnp.float32)
        acc_s[...] = jnp.zeros((H, DH), jnp.float32)
        c_s[...] = jnp.zeros((H, 16), jnp.float32)

    @pl.when(sb < nact)
    def _():
        q_all = q_ref[0]
        asl = asl_ref[0]
        sm = sm_ref[...]

        onehot_b = sm == asl
        onehot = onehot_b.astype(jnp.float32)
        not_a = 1.0 - jnp.max(onehot, axis=0, keepdims=True)

        s_col = sb * SBLK + lax.broadcasted_iota(jnp.int32, (1, SBLK), 1)
        smask = s_col < lenb
        rowmask = (sb * SBLK
                   + lax.broadcasted_iota(jnp.int32, (SBLK, 1), 0)) < lenb

        dn = (((1,), (1,)), ((), ()))
        sc_parts = []
        for h in range(KVH):
            qg = q_all[h * GROUP:(h + 1) * GROUP]
            kh = gk_ref[0, h]
            knh = kn_ref[:, h, :]
            s_c = lax.dot_general(qg, kh, dn,
                                  preferred_element_type=jnp.float32)
            s_n = lax.dot_general(qg, knh, dn,
                                  preferred_element_type=jnp.float32)
            corr = lax.dot_general(
                s_n, onehot, (((1,), (0,)), ((), ())),
                preferred_element_type=jnp.float32)
            sc_parts.append(s_c * not_a + corr)
        scores = jnp.concatenate(sc_parts, axis=0) * SCALE
        scores = jnp.where(smask, scores, NEG)

        m_prev = m_s[...]
        m_new = jnp.maximum(m_prev, jnp.max(scores, axis=1, keepdims=True))
        alpha = jnp.exp(m_prev - m_new)
        p = jnp.exp(scores - m_new)
        l_s[...] = l_s[...] * alpha + jnp.sum(p, axis=1, keepdims=True)
        pmask = p * not_a

        o_parts = []
        c_parts = []
        for h in range(KVH):
            pg = pmask[h * GROUP:(h + 1) * GROUP]
            vh = jnp.where(rowmask, gv_ref[0, h], 0.0)
            o_parts.append(lax.dot_general(
                pg, vh, (((1,), (0,)), ((), ())),
                preferred_element_type=jnp.float32))
            c_parts.append(lax.dot_general(
                p[h * GROUP:(h + 1) * GROUP], onehot, dn,
                preferred_element_type=jnp.float32))
        o_add = jnp.concatenate(o_parts, axis=0)
        c_add = jnp.concatenate(c_parts, axis=0)
        acc_s[...] = acc_s[...] * alpha + o_add
        c_s[...] = c_s[...] * alpha + c_add
        m_s[...] = m_new

    @pl.when(sb == NSB - 1)
    def _():
        c_all = c_s[...]
        on_parts = []
        for h in range(KVH):
            vnh = vn_ref[:, h, :]
            on_parts.append(lax.dot_general(
                c_all[h * GROUP:(h + 1) * GROUP], vnh,
                (((1,), (0,)), ((), ())),
                preferred_element_type=jnp.float32))
        o_new = jnp.concatenate(on_parts, axis=0)
        out_ref[0] = (acc_s[...] + o_new) / l_s[...]


def _tc_attend(q3, kn, vn, sm2, asl3, gk3, gv3, context_lens):
    def clamp(sb, lens, b):
        nact = (jnp.maximum(lens[b], 1) + SBLK - 1) // SBLK
        return jnp.minimum(sb, nact - 1)

    grid_spec = pltpu.PrefetchScalarGridSpec(
        num_scalar_prefetch=1,
        grid=(NBH, NSB),
        in_specs=[
            pl.BlockSpec((1, H, DH), lambda b, sb, lens: (b, 0, 0)),
            pl.BlockSpec((B, KVH, DH), lambda b, sb, lens: (0, 0, 0)),
            pl.BlockSpec((B, KVH, DH), lambda b, sb, lens: (0, 0, 0)),
            pl.BlockSpec((16, 1), lambda b, sb, lens: (0, 0)),
            pl.BlockSpec((1, 1, SBLK),
                         lambda b, sb, lens: (b, 0, clamp(sb, lens, b))),
            pl.BlockSpec((1, KVH, SBLK, DH),
                         lambda b, sb, lens: (b, 0, clamp(sb, lens, b), 0)),
            pl.BlockSpec((1, KVH, SBLK, DH),
                         lambda b, sb, lens: (b, 0, clamp(sb, lens, b), 0)),
        ],
        out_specs=pl.BlockSpec((1, H, DH), lambda b, sb, lens: (b, 0, 0)),
        scratch_shapes=[
            pltpu.VMEM((H, 1), jnp.float32),
            pltpu.VMEM((H, 1), jnp.float32),
            pltpu.VMEM((H, DH), jnp.float32),
            pltpu.VMEM((H, 16), jnp.float32),
        ],
    )
    fn = pl.pallas_call(
        _tc_attend_body,
        grid_spec=grid_spec,
        out_shape=jax.ShapeDtypeStruct((NBH, H, DH), jnp.float32),
        compiler_params=pltpu.CompilerParams(
            dimension_semantics=("arbitrary", "arbitrary")),
    )
    return fn(context_lens, q3, kn, vn, sm2, asl3, gk3, gv3)


def kernel(q, k, v, k_cache, v_cache, slot_mapping, active_slots, context_lens):
    asl_flat = active_slots.reshape(B * S)
    lens_f = context_lens.astype(jnp.float32)
    sm2 = slot_mapping.reshape(16, 1)
    asl3 = active_slots.reshape(B, 1, S)
    gathered = [_sc_gather(b0, k_cache, v_cache, asl_flat, lens_f)
                for b0 in range(0, B, NBH)]
    outs = []
    for i, (gk, gv) in enumerate(gathered):
        b0 = i * NBH
        outs.append(_tc_attend(
            q[b0:b0 + NBH], k, v, sm2, asl3[b0:b0 + NBH], gk, gv,
            context_lens[b0:b0 + NBH]))
    return jnp.concatenate(outs, axis=0)


# speedup vs baseline: 1.1042x; 1.1042x over previous
import functools

import jax
import jax.numpy as jnp
from jax import lax
from jax.experimental import pallas as pl
from jax.experimental.pallas import tpu as pltpu
from jax.experimental.pallas import tpu_sc as plsc

B = 16
S = 2048
H = 32
KVH = 8
DH = 128
SLOTS = 32768
SCALE = 0.08838834764831845
GROUP = H // KVH
ROW = KVH * DH

NTILES = 32
CH = 32
CPB = S // CH
NSPLIT = 4
NBH = B // NSPLIT
NCAND = NBH * CPB // NTILES
CPT = NCAND // NBH
OFF = NTILES // NBH

SBLK = 512
NSB = S // SBLK

NEG = -1e30


def _sc_gather_body(b0, kc_hbm, vc_hbm, asl_hbm, lens_hbm, gk_hbm, gv_hbm,
                    idx_v, kbuf, vbuf, lens_v, sem_k, sem_v, sem_wk, sem_wv):
    t = lax.axis_index("s") * 2 + lax.axis_index("c")
    pltpu.sync_copy(lens_hbm, lens_v)
    lane = lax.iota(jnp.int32, 16)

    def body(i, carry):
        bb = i // CPT
        j = i % CPT
        c = lax.rem(t + NTILES - OFF * bb, NTILES) + NTILES * j
        b = b0 + bb
        lenb = jnp.sum(jnp.where(lane == b, lens_v[...], 0.0), axis=0)
        active = (c == 0) | (lenb > (c * CH).astype(jnp.float32))

        @pl.when(active)
        def _():
            s0 = c * CH
            pltpu.sync_copy(asl_hbm.at[pl.ds(b * S + s0, CH)], idx_v)
            ck = pltpu.async_copy(kc_hbm.at[idx_v], kbuf, sem_k)
            cv = pltpu.async_copy(vc_hbm.at[idx_v], vbuf, sem_v)
            ck.wait()
            wk = [pltpu.async_copy(kbuf.at[:, h, :],
                                   gk_hbm.at[bb, h, pl.ds(s0, CH), :],
                                   sem_wk)
                  for h in range(KVH)]
            cv.wait()
            wv = [pltpu.async_copy(vbuf.at[:, h, :],
                                   gv_hbm.at[bb, h, pl.ds(s0, CH), :],
                                   sem_wv)
                  for h in range(KVH)]
            for w in wk:
                w.wait()
            for w in wv:
                w.wait()

        return carry

    lax.fori_loop(0, NCAND, body, 0)


def _sc_gather(b0, kc, vc, asl_flat, lens_f):
    mesh = plsc.VectorSubcoreMesh(core_axis_name="c", subcore_axis_name="s")
    fn = pl.kernel(
        functools.partial(_sc_gather_body, b0),
        out_type=[jax.ShapeDtypeStruct((NBH, KVH, S, DH), jnp.float32),
                  jax.ShapeDtypeStruct((NBH, KVH, S, DH), jnp.float32)],
        mesh=mesh,
        scratch_types=[
            pltpu.VMEM((CH,), jnp.int32),
            pltpu.VMEM((CH, KVH, DH), jnp.float32),
            pltpu.VMEM((CH, KVH, DH), jnp.float32),
            pltpu.VMEM((B,), jnp.float32),
            pltpu.SemaphoreType.DMA,
            pltpu.SemaphoreType.DMA,
            pltpu.SemaphoreType.DMA,
            pltpu.SemaphoreType.DMA,
        ],
        compiler_params=pltpu.CompilerParams(needs_layout_passes=False),
        name=f"sc_gather_b{b0}",
    )
    return fn(kc, vc, asl_flat, lens_f)


def _tc_attend_body(lens_sref, q_ref, kn_ref, vn_ref, sm_ref, asl_ref,
                    gk_ref, gv_ref, out_ref, m_s, l_s, acc_s, c_s):
    b = pl.program_id(0)
    sb = pl.program_id(1)
    lenb = jnp.maximum(lens_sref[b], 1)
    nact = (lenb + SBLK - 1) // SBLK

    @pl.when(sb == 0)
    def _():
        m_s[...] = jnp.full((H, 1), -3e38, jnp.float32)
        l_s[...] = jnp.zeros((H, 1), jnp.float32)
        acc_s[...] = jnp.zeros((H, DH), jnp.float32)
        c_s[...] = jnp.zeros((H, 16), jnp.float32)

    @pl.when(sb < nact)
    def _():
        q_all = q_ref[0]
        asl = asl_ref[0]
        sm = sm_ref[...]

        onehot_b = sm == asl
        onehot = onehot_b.astype(jnp.float32)
        not_a = 1.0 - jnp.max(onehot, axis=0, keepdims=True)

        s_col = sb * SBLK + lax.broadcasted_iota(jnp.int32, (1, SBLK), 1)
        smask = s_col < lenb
        rowmask = (sb * SBLK
                   + lax.broadcasted_iota(jnp.int32, (SBLK, 1), 0)) < lenb

        dn = (((1,), (1,)), ((), ()))
        sc_parts = []
        for h in range(KVH):
            qg = q_all[h * GROUP:(h + 1) * GROUP]
            kh = gk_ref[0, h]
            knh = kn_ref[:, h, :]
            s_c = lax.dot_general(qg, kh, dn,
                                  preferred_element_type=jnp.float32)
            s_n = lax.dot_general(qg, knh, dn,
                                  preferred_element_type=jnp.float32)
            corr = lax.dot_general(
                s_n, onehot, (((1,), (0,)), ((), ())),
                preferred_element_type=jnp.float32)
            sc_parts.append(s_c * not_a + corr)
        scores = jnp.concatenate(sc_parts, axis=0) * SCALE
        scores = jnp.where(smask, scores, NEG)

        m_prev = m_s[...]
        m_new = jnp.maximum(m_prev, jnp.max(scores, axis=1, keepdims=True))
        alpha = jnp.exp(m_prev - m_new)
        p = jnp.exp(scores - m_new)
        l_s[...] = l_s[...] * alpha + jnp.sum(p, axis=1, keepdims=True)
        pmask = p * not_a

        o_parts = []
        c_parts = []
        for h in range(KVH):
            pg = pmask[h * GROUP:(h + 1) * GROUP]
            vh = jnp.where(rowmask, gv_ref[0, h], 0.0)
            o_parts.append(lax.dot_general(
                pg, vh, (((1,), (0,)), ((), ())),
                preferred_element_type=jnp.float32))
            c_parts.append(lax.dot_general(
                p[h * GROUP:(h + 1) * GROUP], onehot, dn,
                preferred_element_type=jnp.float32))
        o_add = jnp.concatenate(o_parts, axis=0)
        c_add = jnp.concatenate(c_parts, axis=0)
        acc_s[...] = acc_s[...] * alpha + o_add
        c_s[...] = c_s[...] * alpha + c_add
        m_s[...] = m_new

    @pl.when(sb == NSB - 1)
    def _():
        c_all = c_s[...]
        on_parts = []
        for h in range(KVH):
            vnh = vn_ref[:, h, :]
            on_parts.append(lax.dot_general(
                c_all[h * GROUP:(h + 1) * GROUP], vnh,
                (((1,), (0,)), ((), ())),
                preferred_element_type=jnp.float32))
        o_new = jnp.concatenate(on_parts, axis=0)
        out_ref[0] = (acc_s[...] + o_new) / l_s[...]


def _tc_attend(q3, kn, vn, sm2, asl3, gk3, gv3, context_lens):
    def clamp(sb, lens, b):
        nact = (jnp.maximum(lens[b], 1) + SBLK - 1) // SBLK
        return jnp.minimum(sb, nact - 1)

    grid_spec = pltpu.PrefetchScalarGridSpec(
        num_scalar_prefetch=1,
        grid=(NBH, NSB),
        in_specs=[
            pl.BlockSpec((1, H, DH), lambda b, sb, lens: (b, 0, 0)),
            pl.BlockSpec((B, KVH, DH), lambda b, sb, lens: (0, 0, 0)),
            pl.BlockSpec((B, KVH, DH), lambda b, sb, lens: (0, 0, 0)),
            pl.BlockSpec((16, 1), lambda b, sb, lens: (0, 0)),
            pl.BlockSpec((1, 1, SBLK),
                         lambda b, sb, lens: (b, 0, clamp(sb, lens, b))),
            pl.BlockSpec((1, KVH, SBLK, DH),
                         lambda b, sb, lens: (b, 0, clamp(sb, lens, b), 0)),
            pl.BlockSpec((1, KVH, SBLK, DH),
                         lambda b, sb, lens: (b, 0, clamp(sb, lens, b), 0)),
        ],
        out_specs=pl.BlockSpec((1, H, DH), lambda b, sb, lens: (b, 0, 0)),
        scratch_shapes=[
            pltpu.VMEM((H, 1), jnp.float32),
            pltpu.VMEM((H, 1), jnp.float32),
            pltpu.VMEM((H, DH), jnp.float32),
            pltpu.VMEM((H, 16), jnp.float32),
        ],
    )
    fn = pl.pallas_call(
        _tc_attend_body,
        grid_spec=grid_spec,
        out_shape=jax.ShapeDtypeStruct((NBH, H, DH), jnp.float32),
        compiler_params=pltpu.CompilerParams(
            dimension_semantics=("arbitrary", "arbitrary")),
    )
    return fn(context_lens, q3, kn, vn, sm2, asl3, gk3, gv3)


def kernel(q, k, v, k_cache, v_cache, slot_mapping, active_slots, context_lens):
    asl_flat = active_slots.reshape(B * S)
    lens_f = context_lens.astype(jnp.float32)
    sm2 = slot_mapping.reshape(16, 1)
    asl3 = active_slots.reshape(B, 1, S)
    gathered = [_sc_gather(b0, k_cache, v_cache, asl_flat, lens_f)
                for b0 in range(0, B, NBH)]
    outs = []
    for i, (gk, gv) in enumerate(gathered):
        b0 = i * NBH
        outs.append(_tc_attend(
            q[b0:b0 + NBH], k, v, sm2, asl3[b0:b0 + NBH], gk, gv,
            context_lens[b0:b0 + NBH]))
    return jnp.concatenate(outs, axis=0)


# speedup vs baseline: 1.1509x; 1.0423x over previous
import functools

import jax
import jax.numpy as jnp
from jax import lax
from jax.experimental import pallas as pl
from jax.experimental.pallas import tpu as pltpu
from jax.experimental.pallas import tpu_sc as plsc

B = 16
S = 2048
H = 32
KVH = 8
DH = 128
SLOTS = 32768
SCALE = 0.08838834764831845
GROUP = H // KVH
ROW = KVH * DH

NTILES = 32
CH = 32
CPB = S // CH
NSPLIT = 2
NBH = B // NSPLIT
NCAND = NBH * CPB // NTILES
CPT = NCAND // NBH
OFF = NTILES // NBH

SBLK = 512
NSB = S // SBLK

NEG = -1e30


def _sc_gather_body(b0, kc_hbm, vc_hbm, asl_hbm, lens_hbm, gk_hbm, gv_hbm,
                    idx_v, kbuf, vbuf, lens_v, sem_k, sem_v, sem_wk, sem_wv):
    t = lax.axis_index("s") * 2 + lax.axis_index("c")
    pltpu.sync_copy(lens_hbm, lens_v)
    lane = lax.iota(jnp.int32, 16)

    def body(i, carry):
        bb = i // CPT
        j = i % CPT
        c = lax.rem(t + NTILES - OFF * bb, NTILES) + NTILES * j
        b = b0 + bb
        lenb = jnp.sum(jnp.where(lane == b, lens_v[...], 0.0), axis=0)
        active = (c == 0) | (lenb > (c * CH).astype(jnp.float32))

        @pl.when(active)
        def _():
            s0 = c * CH
            pltpu.sync_copy(asl_hbm.at[pl.ds(b * S + s0, CH)], idx_v)
            ck = pltpu.async_copy(kc_hbm.at[idx_v], kbuf, sem_k)
            cv = pltpu.async_copy(vc_hbm.at[idx_v], vbuf, sem_v)
            ck.wait()
            wk = [pltpu.async_copy(kbuf.at[:, h, :],
                                   gk_hbm.at[bb, h, pl.ds(s0, CH), :],
                                   sem_wk)
                  for h in range(KVH)]
            cv.wait()
            wv = [pltpu.async_copy(vbuf.at[:, h, :],
                                   gv_hbm.at[bb, h, pl.ds(s0, CH), :],
                                   sem_wv)
                  for h in range(KVH)]
            for w in wk:
                w.wait()
            for w in wv:
                w.wait()

        return carry

    lax.fori_loop(0, NCAND, body, 0)


def _sc_gather(b0, kc, vc, asl_flat, lens_f):
    mesh = plsc.VectorSubcoreMesh(core_axis_name="c", subcore_axis_name="s")
    fn = pl.kernel(
        functools.partial(_sc_gather_body, b0),
        out_type=[jax.ShapeDtypeStruct((NBH, KVH, S, DH), jnp.float32),
                  jax.ShapeDtypeStruct((NBH, KVH, S, DH), jnp.float32)],
        mesh=mesh,
        scratch_types=[
            pltpu.VMEM((CH,), jnp.int32),
            pltpu.VMEM((CH, KVH, DH), jnp.float32),
            pltpu.VMEM((CH, KVH, DH), jnp.float32),
            pltpu.VMEM((B,), jnp.float32),
            pltpu.SemaphoreType.DMA,
            pltpu.SemaphoreType.DMA,
            pltpu.SemaphoreType.DMA,
            pltpu.SemaphoreType.DMA,
        ],
        compiler_params=pltpu.CompilerParams(needs_layout_passes=False),
        name=f"sc_gather_b{b0}",
    )
    return fn(kc, vc, asl_flat, lens_f)


def _tc_attend_body(lens_sref, q_ref, kn_ref, vn_ref, sm_ref, asl_ref,
                    gk_ref, gv_ref, out_ref, m_s, l_s, acc_s, c_s):
    b = pl.program_id(0)
    sb = pl.program_id(1)
    lenb = jnp.maximum(lens_sref[b], 1)
    nact = (lenb + SBLK - 1) // SBLK

    @pl.when(sb == 0)
    def _():
        m_s[...] = jnp.full((H, 1), -3e38, jnp.float32)
        l_s[...] = jnp.zeros((H, 1), jnp.float32)
        acc_s[...] = jnp.zeros((H, DH), jnp.float32)
        c_s[...] = jnp.zeros((H, 16), jnp.float32)

    @pl.when(sb < nact)
    def _():
        q_all = q_ref[0]
        asl = asl_ref[0]
        sm = sm_ref[...]

        onehot_b = sm == asl
        onehot = onehot_b.astype(jnp.float32)
        not_a = 1.0 - jnp.max(onehot, axis=0, keepdims=True)

        s_col = sb * SBLK + lax.broadcasted_iota(jnp.int32, (1, SBLK), 1)
        smask = s_col < lenb
        rowmask = (sb * SBLK
                   + lax.broadcasted_iota(jnp.int32, (SBLK, 1), 0)) < lenb

        dn = (((1,), (1,)), ((), ()))
        sc_parts = []
        for h in range(KVH):
            qg = q_all[h * GROUP:(h + 1) * GROUP]
            kh = gk_ref[0, h]
            knh = kn_ref[:, h, :]
            s_c = lax.dot_general(qg, kh, dn,
                                  preferred_element_type=jnp.float32)
            s_n = lax.dot_general(qg, knh, dn,
                                  preferred_element_type=jnp.float32)
            corr = lax.dot_general(
                s_n, onehot, (((1,), (0,)), ((), ())),
                preferred_element_type=jnp.float32)
            sc_parts.append(s_c * not_a + corr)
        scores = jnp.concatenate(sc_parts, axis=0) * SCALE
        scores = jnp.where(smask, scores, NEG)

        m_prev = m_s[...]
        m_new = jnp.maximum(m_prev, jnp.max(scores, axis=1, keepdims=True))
        alpha = jnp.exp(m_prev - m_new)
        p = jnp.exp(scores - m_new)
        l_s[...] = l_s[...] * alpha + jnp.sum(p, axis=1, keepdims=True)
        pmask = p * not_a

        o_parts = []
        c_parts = []
        for h in range(KVH):
            pg = pmask[h * GROUP:(h + 1) * GROUP]
            vh = jnp.where(rowmask, gv_ref[0, h], 0.0)
            o_parts.append(lax.dot_general(
                pg, vh, (((1,), (0,)), ((), ())),
                preferred_element_type=jnp.float32))
            c_parts.append(lax.dot_general(
                p[h * GROUP:(h + 1) * GROUP], onehot, dn,
                preferred_element_type=jnp.float32))
        o_add = jnp.concatenate(o_parts, axis=0)
        c_add = jnp.concatenate(c_parts, axis=0)
        acc_s[...] = acc_s[...] * alpha + o_add
        c_s[...] = c_s[...] * alpha + c_add
        m_s[...] = m_new

    @pl.when(sb == NSB - 1)
    def _():
        c_all = c_s[...]
        on_parts = []
        for h in range(KVH):
            vnh = vn_ref[:, h, :]
            on_parts.append(lax.dot_general(
                c_all[h * GROUP:(h + 1) * GROUP], vnh,
                (((1,), (0,)), ((), ())),
                preferred_element_type=jnp.float32))
        o_new = jnp.concatenate(on_parts, axis=0)
        out_ref[0] = (acc_s[...] + o_new) / l_s[...]


def _tc_attend(q3, kn, vn, sm2, asl3, gk3, gv3, context_lens):
    def clamp(sb, lens, b):
        nact = (jnp.maximum(lens[b], 1) + SBLK - 1) // SBLK
        return jnp.minimum(sb, nact - 1)

    grid_spec = pltpu.PrefetchScalarGridSpec(
        num_scalar_prefetch=1,
        grid=(NBH, NSB),
        in_specs=[
            pl.BlockSpec((1, H, DH), lambda b, sb, lens: (b, 0, 0)),
            pl.BlockSpec((B, KVH, DH), lambda b, sb, lens: (0, 0, 0)),
            pl.BlockSpec((B, KVH, DH), lambda b, sb, lens: (0, 0, 0)),
            pl.BlockSpec((16, 1), lambda b, sb, lens: (0, 0)),
            pl.BlockSpec((1, 1, SBLK),
                         lambda b, sb, lens: (b, 0, clamp(sb, lens, b))),
            pl.BlockSpec((1, KVH, SBLK, DH),
                         lambda b, sb, lens: (b, 0, clamp(sb, lens, b), 0)),
            pl.BlockSpec((1, KVH, SBLK, DH),
                         lambda b, sb, lens: (b, 0, clamp(sb, lens, b), 0)),
        ],
        out_specs=pl.BlockSpec((1, H, DH), lambda b, sb, lens: (b, 0, 0)),
        scratch_shapes=[
            pltpu.VMEM((H, 1), jnp.float32),
            pltpu.VMEM((H, 1), jnp.float32),
            pltpu.VMEM((H, DH), jnp.float32),
            pltpu.VMEM((H, 16), jnp.float32),
        ],
    )
    fn = pl.pallas_call(
        _tc_attend_body,
        grid_spec=grid_spec,
        out_shape=jax.ShapeDtypeStruct((NBH, H, DH), jnp.float32),
        compiler_params=pltpu.CompilerParams(
            dimension_semantics=("arbitrary", "arbitrary")),
    )
    return fn(context_lens, q3, kn, vn, sm2, asl3, gk3, gv3)


def kernel(q, k, v, k_cache, v_cache, slot_mapping, active_slots, context_lens):
    asl_flat = active_slots.reshape(B * S)
    lens_f = context_lens.astype(jnp.float32)
    sm2 = slot_mapping.reshape(16, 1)
    asl3 = active_slots.reshape(B, 1, S)
    gathered = [_sc_gather(b0, k_cache, v_cache, asl_flat, lens_f)
                for b0 in range(0, B, NBH)]
    outs = []
    for i, (gk, gv) in enumerate(gathered):
        b0 = i * NBH
        outs.append(_tc_attend(
            q[b0:b0 + NBH], k, v, sm2, asl3[b0:b0 + NBH], gk, gv,
            context_lens[b0:b0 + NBH]))
    return jnp.concatenate(outs, axis=0)
